# baseline (speedup 1.0000x reference)
import functools

import jax
import jax.numpy as jnp
from jax.experimental import pallas as pl
from jax.experimental.pallas import tpu as pltpu
from jax.experimental.pallas import tpu_sc as plsc

E = 8
D = 1024
H = 400
O = 1024
B = 4096
T = 256
NT = B // T
S = NT + E - 1
BP = S * T

_VEC_MESH = plsc.VectorSubcoreMesh(core_axis_name="c", subcore_axis_name="s")
_NC, _NS = 2, 16
_NW = _NC * _NS
_CHUNK = 64

_RB = 128
_NRB = B // _RB


def _sc_scatter_rows(x, idx, out_rows):
    n, d = x.shape
    b_per_w = n // _NW

    @functools.partial(pl.kernel,
                       out_type=jax.ShapeDtypeStruct((out_rows, d), x.dtype),
                       mesh=_VEC_MESH,
                       scratch_types=[pltpu.VMEM((_CHUNK,), jnp.int32),
                                      pltpu.VMEM((_CHUNK, d), x.dtype),
                                      pltpu.SemaphoreType.DMA])
    def scatter_kernel(x_hbm, i_hbm, o_hbm, idx_v, rows_v, sem):
        wid = jax.lax.axis_index("s") * _NC + jax.lax.axis_index("c")
        base = wid * b_per_w

        @pl.loop(0, b_per_w // _CHUNK)
        def _(c):
            off = base + c * _CHUNK
            pltpu.sync_copy(i_hbm.at[pl.ds(off, _CHUNK)], idx_v)
            pltpu.sync_copy(x_hbm.at[pl.ds(off, _CHUNK)], rows_v)
            pltpu.async_copy(rows_v, o_hbm.at[idx_v], sem).wait()

    return scatter_kernel(x, idx)


def _sc_gather_rows(x, idx):
    n, d = x.shape
    m = idx.shape[0]
    b_per_w = m // _NW

    @functools.partial(pl.kernel,
                       out_type=jax.ShapeDtypeStruct((m, d), x.dtype),
                       mesh=_VEC_MESH,
                       scratch_types=[pltpu.VMEM((_CHUNK,), jnp.int32),
                                      pltpu.VMEM((_CHUNK, d), x.dtype),
                                      pltpu.SemaphoreType.DMA])
    def gather_kernel(x_hbm, i_hbm, o_hbm, idx_v, rows_v, sem):
        wid = jax.lax.axis_index("s") * _NC + jax.lax.axis_index("c")
        base = wid * b_per_w

        @pl.loop(0, b_per_w // _CHUNK)
        def _(c):
            off = base + c * _CHUNK
            pltpu.sync_copy(i_hbm.at[pl.ds(off, _CHUNK)], idx_v)
            pltpu.async_copy(x_hbm.at[idx_v], rows_v, sem).wait()
            pltpu.sync_copy(rows_v, o_hbm.at[pl.ds(off, _CHUNK)])

    return gather_kernel(x, idx)


def _mlp_body(te_ref, z_ref, w1_ref, b1_ref, w2_ref, b2_ref, out_ref):
    x = z_ref[...]
    h = jax.lax.dot_general(x, w1_ref[0], (((1,), (0,)), ((), ())),
                            preferred_element_type=jnp.float32)
    h = jnp.maximum(h + b1_ref[0], 0.0)
    y = jax.lax.dot_general(h, w2_ref[0], (((1,), (0,)), ((), ())),
                            preferred_element_type=jnp.float32)
    out_ref[...] = jax.nn.sigmoid(y + b2_ref[0])


def _grouped_mlp(z_padded, W1, b1, W2, b2, te):
    grid_spec = pltpu.PrefetchScalarGridSpec(
        num_scalar_prefetch=1,
        grid=(S,),
        in_specs=[
            pl.BlockSpec((T, D), lambda s, te: (s, 0)),
            pl.BlockSpec((1, D, H), lambda s, te: (te[s], 0, 0)),
            pl.BlockSpec((1, 1, H), lambda s, te: (te[s], 0, 0)),
            pl.BlockSpec((1, H, O), lambda s, te: (te[s], 0, 0)),
            pl.BlockSpec((1, 1, O), lambda s, te: (te[s], 0, 0)),
        ],
        out_specs=pl.BlockSpec((T, O), lambda s, te: (s, 0)),
    )
    return pl.pallas_call(
        _mlp_body,
        grid_spec=grid_spec,
        out_shape=jax.ShapeDtypeStruct((BP, O), jnp.float32),
    )(te, z_padded, W1, b1, W2, b2)


def kernel(z, angle_idx, W1, b1, W2, b2):
    e32 = angle_idx.astype(jnp.int32)
    onehot = (e32[:, None] == jnp.arange(E, dtype=jnp.int32)[None, :])

    oh3 = onehot.astype(jnp.bfloat16).reshape(_NRB, _RB, E)
    tril = jnp.tril(jnp.ones((_RB, _RB), jnp.bfloat16))
    within = jnp.einsum('ij,bjk->bik', tril, oh3,
                        preferred_element_type=jnp.float32)
    bsum = within[:, _RB - 1, :]
    bpref = (jnp.cumsum(bsum, axis=0) - bsum)[:, None, :]
    ranks = (within + bpref).reshape(B, E)
    rank_in_e = jnp.sum(jnp.where(onehot, ranks, 0), axis=1
                        ).astype(jnp.int32) - 1
    counts = (bsum.sum(axis=0)).astype(jnp.int32)

    ptiles = (counts + (T - 1)) // T
    gtile = jnp.cumsum(ptiles) - ptiles
    gstart = gtile * T
    pos = jnp.sum(jnp.where(onehot, gstart[None, :], 0), axis=1) + rank_in_e
    te = jnp.clip(jnp.searchsorted(gtile, jnp.arange(S, dtype=jnp.int32),
                                   side="right") - 1, 0, E - 1).astype(jnp.int32)

    z_padded = _sc_scatter_rows(z, pos.astype(jnp.int32), BP)
    y_padded = _grouped_mlp(z_padded, W1.astype(jnp.bfloat16),
                            b1.reshape(E, 1, H), W2.astype(jnp.bfloat16),
                            b2.reshape(E, 1, O), te)
    return _sc_gather_rows(y_padded, pos.astype(jnp.int32))


# speedup vs baseline: 1.1144x; 1.1144x over previous
import functools

import jax
import jax.numpy as jnp
from jax.experimental import pallas as pl
from jax.experimental.pallas import tpu as pltpu
from jax.experimental.pallas import tpu_sc as plsc

E = 8
D = 1024
H = 400
O = 1024
B = 4096
T = 256
NT = B // T
S = NT + E - 1
BP = S * T

_VEC_MESH = plsc.VectorSubcoreMesh(core_axis_name="c", subcore_axis_name="s")
_NC, _NS = 2, 16
_NW = _NC * _NS
_CHUNK = 64

_RB = 128
_NRB = B // _RB


def _sc_scatter_rows(x, idx, out_rows):
    n, d = x.shape
    b_per_w = n // _NW

    @functools.partial(pl.kernel,
                       out_type=jax.ShapeDtypeStruct((out_rows, d), x.dtype),
                       mesh=_VEC_MESH,
                       scratch_types=[pltpu.VMEM((_CHUNK,), jnp.int32),
                                      pltpu.VMEM((_CHUNK, d), x.dtype),
                                      pltpu.SemaphoreType.DMA])
    def scatter_kernel(x_hbm, i_hbm, o_hbm, idx_v, rows_v, sem):
        wid = jax.lax.axis_index("s") * _NC + jax.lax.axis_index("c")
        base = wid * b_per_w

        @pl.loop(0, b_per_w // _CHUNK)
        def _(c):
            off = base + c * _CHUNK
            pltpu.sync_copy(i_hbm.at[pl.ds(off, _CHUNK)], idx_v)
            pltpu.sync_copy(x_hbm.at[pl.ds(off, _CHUNK)], rows_v)
            pltpu.async_copy(rows_v, o_hbm.at[idx_v], sem).wait()

    return scatter_kernel(x, idx)


def _sc_gather_rows(x, idx):
    n, d = x.shape
    m = idx.shape[0]
    b_per_w = m // _NW

    @functools.partial(pl.kernel,
                       out_type=jax.ShapeDtypeStruct((m, d), x.dtype),
                       mesh=_VEC_MESH,
                       scratch_types=[pltpu.VMEM((_CHUNK,), jnp.int32),
                                      pltpu.VMEM((_CHUNK, d), x.dtype),
                                      pltpu.SemaphoreType.DMA])
    def gather_kernel(x_hbm, i_hbm, o_hbm, idx_v, rows_v, sem):
        wid = jax.lax.axis_index("s") * _NC + jax.lax.axis_index("c")
        base = wid * b_per_w

        @pl.loop(0, b_per_w // _CHUNK)
        def _(c):
            off = base + c * _CHUNK
            pltpu.sync_copy(i_hbm.at[pl.ds(off, _CHUNK)], idx_v)
            pltpu.async_copy(x_hbm.at[idx_v], rows_v, sem).wait()
            pltpu.sync_copy(rows_v, o_hbm.at[pl.ds(off, _CHUNK)])

    return gather_kernel(x, idx)


def _route_body(e_ref, pos_ref, te_ref):
    ei = e_ref[...]
    rows, lanes = ei.shape
    li = jax.lax.broadcasted_iota(jnp.int32, (lanes, lanes), 0)
    lj = jax.lax.broadcasted_iota(jnp.int32, (lanes, lanes), 1)
    U = (li <= lj).astype(jnp.bfloat16)
    J = jnp.ones((lanes, lanes), jnp.bfloat16)
    ri = jax.lax.broadcasted_iota(jnp.int32, (rows, rows), 0)
    rj = jax.lax.broadcasted_iota(jnp.int32, (rows, rows), 1)
    A = (rj < ri).astype(jnp.bfloat16)
    dn = (((1,), (0,)), ((), ()))

    pos_acc = jnp.zeros((rows, lanes), jnp.float32)
    gtile = jnp.int32(0)
    te_acc = jnp.zeros((1, 32), jnp.int32)
    si = jax.lax.broadcasted_iota(jnp.int32, (1, 32), 1)
    for e in range(E):
        m = (ei == e).astype(jnp.bfloat16)
        lane_pref = jax.lax.dot_general(m, U, dn,
                                        preferred_element_type=jnp.float32)
        row_tot = jax.lax.dot_general(m, J, dn,
                                      preferred_element_type=jnp.float32)
        prev_rows = jax.lax.dot_general(A, row_tot, dn,
                                        preferred_element_type=jnp.float32)
        incl = lane_pref + prev_rows
        count = jnp.sum(row_tot[:, 0].astype(jnp.int32))
        pos_acc = pos_acc + m.astype(jnp.float32) * (
            incl + (gtile * T).astype(jnp.float32))
        te_acc = te_acc + jnp.where(si >= gtile, 1, 0)
        gtile = gtile + (count + (T - 1)) // T
    pos_ref[...] = pos_acc.astype(jnp.int32) - 1
    te_ref[...] = jnp.clip(te_acc - 1, 0, E - 1)


def _route(e2d):
    return pl.pallas_call(
        _route_body,
        grid=(1,),
        in_specs=[pl.BlockSpec((32, 128), lambda s: (0, 0))],
        out_specs=[pl.BlockSpec((32, 128), lambda s: (0, 0)),
                   pl.BlockSpec((1, 32), lambda s: (0, 0))],
        out_shape=[jax.ShapeDtypeStruct((32, 128), jnp.int32),
                   jax.ShapeDtypeStruct((1, 32), jnp.int32)],
    )(e2d)


def _mlp_body(te_ref, z_ref, w1_ref, b1_ref, w2_ref, b2_ref, out_ref):
    x = z_ref[...]
    h = jax.lax.dot_general(x, w1_ref[0], (((1,), (0,)), ((), ())),
                            preferred_element_type=jnp.float32)
    h = jnp.maximum(h + b1_ref[0], 0.0)
    y = jax.lax.dot_general(h, w2_ref[0], (((1,), (0,)), ((), ())),
                            preferred_element_type=jnp.float32)
    out_ref[...] = jax.nn.sigmoid(y + b2_ref[0])


def _grouped_mlp(z_padded, W1, b1, W2, b2, te):
    grid_spec = pltpu.PrefetchScalarGridSpec(
        num_scalar_prefetch=1,
        grid=(S,),
        in_specs=[
            pl.BlockSpec((T, D), lambda s, te: (s, 0)),
            pl.BlockSpec((1, D, H), lambda s, te: (te[s], 0, 0)),
            pl.BlockSpec((1, 1, H), lambda s, te: (te[s], 0, 0)),
            pl.BlockSpec((1, H, O), lambda s, te: (te[s], 0, 0)),
            pl.BlockSpec((1, 1, O), lambda s, te: (te[s], 0, 0)),
        ],
        out_specs=pl.BlockSpec((T, O), lambda s, te: (s, 0)),
    )
    return pl.pallas_call(
        _mlp_body,
        grid_spec=grid_spec,
        out_shape=jax.ShapeDtypeStruct((BP, O), jnp.float32),
    )(te, z_padded, W1, b1, W2, b2)


def kernel(z, angle_idx, W1, b1, W2, b2):
    e2d = angle_idx.astype(jnp.int32).reshape(32, 128)
    pos2d, te2d = _route(e2d)
    pos = pos2d.reshape(B)
    te = te2d.reshape(32)

    z_padded = _sc_scatter_rows(z, pos, BP)
    y_padded = _grouped_mlp(z_padded, W1, b1.reshape(E, 1, H), W2,
                            b2.reshape(E, 1, O), te)
    return _sc_gather_rows(y_padded, pos)
